# baseline (speedup 1.0000x reference)
import functools
import jax
import jax.numpy as jnp
from jax import lax
from jax.experimental import pallas as pl
from jax.experimental.pallas import tpu as pltpu
from jax.experimental.pallas import tpu_sc as plsc

N = 10000
D = 128
NPAD = 10240
NC = 2
NS = 16
NW = NC * NS
KE = 128
KG = 128
BM = 512
R = NPAD // BM
RPT = NPAD // NS

_f32 = jnp.float32
_i32 = jnp.int32


def _mesh():
    return plsc.VectorSubcoreMesh(
        core_axis_name="c", subcore_axis_name="s", num_cores=NC, num_subcores=NS
    )


def _dense_first(x, W, att_src, att_dst):
    def body(x_ref, w_ref, asv_ref, adv_ref, h_ref, aso_ref, ado_ref):
        h = jnp.dot(x_ref[...], w_ref[...], preferred_element_type=_f32)
        h_ref[...] = h
        aso_ref[0, 0, :] = jnp.sum(h * asv_ref[0, :][None, :], axis=1)
        ado_ref[0, 0, :] = jnp.sum(h * adv_ref[0, :][None, :], axis=1)

    return pl.pallas_call(
        body,
        grid=(R,),
        in_specs=[
            pl.BlockSpec((BM, D), lambda i: (i, 0)),
            pl.BlockSpec((D, D), lambda i: (0, 0)),
            pl.BlockSpec((1, D), lambda i: (0, 0)),
            pl.BlockSpec((1, D), lambda i: (0, 0)),
        ],
        out_specs=[
            pl.BlockSpec((BM, D), lambda i: (i, 0)),
            pl.BlockSpec((1, 1, BM), lambda i: (i, 0, 0)),
            pl.BlockSpec((1, 1, BM), lambda i: (i, 0, 0)),
        ],
        out_shape=[
            jax.ShapeDtypeStruct((NPAD, D), _f32),
            jax.ShapeDtypeStruct((R, 1, BM), _f32),
            jax.ShapeDtypeStruct((R, 1, BM), _f32),
        ],
    )(x, W, att_src, att_dst)


def _dense_norm(num, s, W, att_src, att_dst, bias):
    def body(num_ref, s_ref, w_ref, asv_ref, adv_ref, b_ref,
             h_ref, aso_ref, ado_ref):
        acc = num_ref[0] + num_ref[1]
        den = s_ref[0, 0, :] + 1e-16
        x = acc / den[:, None] + b_ref[0, :][None, :]
        h = jnp.dot(x, w_ref[...], preferred_element_type=_f32)
        h_ref[...] = h
        aso_ref[0, 0, :] = jnp.sum(h * asv_ref[0, :][None, :], axis=1)
        ado_ref[0, 0, :] = jnp.sum(h * adv_ref[0, :][None, :], axis=1)

    return pl.pallas_call(
        body,
        grid=(R,),
        in_specs=[
            pl.BlockSpec((NC, BM, D), lambda i: (0, i, 0)),
            pl.BlockSpec((1, 1, BM), lambda i: (i, 0, 0)),
            pl.BlockSpec((D, D), lambda i: (0, 0)),
            pl.BlockSpec((1, D), lambda i: (0, 0)),
            pl.BlockSpec((1, D), lambda i: (0, 0)),
            pl.BlockSpec((1, D), lambda i: (0, 0)),
        ],
        out_specs=[
            pl.BlockSpec((BM, D), lambda i: (i, 0)),
            pl.BlockSpec((1, 1, BM), lambda i: (i, 0, 0)),
            pl.BlockSpec((1, 1, BM), lambda i: (i, 0, 0)),
        ],
        out_shape=[
            jax.ShapeDtypeStruct((NPAD, D), _f32),
            jax.ShapeDtypeStruct((R, 1, BM), _f32),
            jax.ShapeDtypeStruct((R, 1, BM), _f32),
        ],
    )(num, s, W, att_src, att_dst, bias)


def _norm_only(num, s, bias):
    def body(num_ref, s_ref, b_ref, x_ref):
        acc = num_ref[0] + num_ref[1]
        den = s_ref[0, 0, :] + 1e-16
        x_ref[...] = acc / den[:, None] + b_ref[0, :][None, :]

    return pl.pallas_call(
        body,
        grid=(R,),
        in_specs=[
            pl.BlockSpec((NC, BM, D), lambda i: (0, i, 0)),
            pl.BlockSpec((1, 1, BM), lambda i: (i, 0, 0)),
            pl.BlockSpec((1, D), lambda i: (0, 0)),
        ],
        out_specs=pl.BlockSpec((BM, D), lambda i: (i, 0)),
        out_shape=jax.ShapeDtypeStruct((NPAD, D), _f32),
    )(num, s, bias)


def _edge_stage(pck2d, a_s, a_d, h, n_blk):
    ngrp = n_blk // 4

    @functools.partial(
        pl.kernel,
        out_type=(
            jax.ShapeDtypeStruct((NC, NPAD, D), _f32),
            jax.ShapeDtypeStruct((NC, NPAD), _f32),
        ),
        mesh=_mesh(),
        compiler_params=pltpu.CompilerParams(needs_layout_passes=False),
        scratch_types=[
            pltpu.VMEM((n_blk, KE), _i32),
            pltpu.VMEM((4, KE), _i32),
            pltpu.VMEM((4, KE), _i32),
            pltpu.VMEM((2, KE), _f32),
            pltpu.VMEM((2, KE), _f32),
            pltpu.VMEM((2, KE), _f32),
            pltpu.VMEM((2, KE, D), _f32),
            pltpu.VMEM((RPT,), _f32),
            pltpu.VMEM_SHARED((NPAD, D), _f32),
            pltpu.VMEM_SHARED((NPAD,), _f32),
            pltpu.VMEM_SHARED((NPAD,), _f32),
            pltpu.VMEM_SHARED((NPAD,), _f32),
            pltpu.SemaphoreType.DMA,
            pltpu.SemaphoreType.DMA,
            pltpu.SemaphoreType.DMA,
            pltpu.SemaphoreType.DMA,
            pltpu.SemaphoreType.DMA,
            pltpu.SemaphoreType.DMA,
            pltpu.SemaphoreType.DMA,
            pltpu.SemaphoreType.DMA,
        ],
    )
    def k(pck_hbm, as_hbm, ad_hbm, h_hbm, num_out, s_out,
          pck_v, sidw, didw, asg, adg, wv, rows, svec,
          num_sh, s_sh, as_sh, ad_sh,
          asem0, asem1, gsem0, gsem1, ssem0, ssem1, dsem0, dsem1):
        asem = (asem0, asem1)
        gsem = (gsem0, gsem1)
        ssem = (ssem0, ssem1)
        dsem = (dsem0, dsem1)
        cid = lax.axis_index("c")
        sid = lax.axis_index("s")
        wid = sid * NC + cid
        row0 = sid * RPT

        pltpu.sync_copy(pck_hbm.at[wid], pck_v)
        pltpu.sync_copy(as_hbm.at[pl.ds(row0, RPT)], svec)
        pltpu.sync_copy(svec, as_sh.at[pl.ds(row0, RPT)])
        pltpu.sync_copy(ad_hbm.at[pl.ds(row0, RPT)], svec)
        pltpu.sync_copy(svec, ad_sh.at[pl.ds(row0, RPT)])

        zeros16 = jnp.zeros((16,), _f32)
        for r0 in range(64):
            for c0 in range(D // 16):
                rows[0, r0, pl.ds(c0 * 16, 16)] = zeros16
        for c0 in range(RPT // 16):
            svec[pl.ds(c0 * 16, 16)] = zeros16

        def zloop(i, _):
            pltpu.sync_copy(rows.at[0, pl.ds(0, 64)],
                            num_sh.at[pl.ds(row0 + i * 64, 64)])
            return 0

        lax.fori_loop(0, RPT // 64, zloop, 0)
        pltpu.sync_copy(svec, s_sh.at[pl.ds(row0, RPT)])
        plsc.subcore_barrier()

        def unpack(bb, slot):
            for c2 in range(KE // 16):
                sl = pl.ds(c2 * 16, 16)
                wrd = pck_v[bb, sl]
                sidw[slot, sl] = jnp.bitwise_and(wrd, 0xFFFF)
                didw[slot, sl] = lax.shift_right_logical(wrd, 16)

        for p in (0, 1):
            unpack(p, p)
            pltpu.async_copy(as_sh.at[sidw.at[p]], asg.at[p], asem[p])
            pltpu.async_copy(ad_sh.at[didw.at[p]], adg.at[p], asem[p])
        pltpu.async_copy(h_hbm.at[sidw.at[0]], rows.at[0], gsem[0])

        def step(g, _):
            for p in range(4):
                b = 4 * g + p
                r = p % 2
                rq = 1 - r
                sp2 = (p + 2) % 4

                pltpu.make_async_copy(
                    as_sh.at[sidw.at[p]], asg.at[r], asem[r]).wait()
                pltpu.make_async_copy(
                    ad_sh.at[didw.at[p]], adg.at[r], asem[r]).wait()
                for c2 in range(KE // 16):
                    sl = pl.ds(c2 * 16, 16)
                    e = asg[r, sl] + adg[r, sl]
                    e = jnp.where(e >= 0.0, e, 0.2 * e)
                    wv[r, sl] = jnp.exp(e)
                if p < 2:
                    unpack(b + 2, sp2)
                    pltpu.async_copy(as_sh.at[sidw.at[sp2]], asg.at[r],
                                     asem[r])
                    pltpu.async_copy(ad_sh.at[didw.at[sp2]], adg.at[r],
                                     asem[r])
                else:
                    @pl.when(g < ngrp - 1)
                    def _():
                        unpack(b + 2, sp2)
                        pltpu.async_copy(as_sh.at[sidw.at[sp2]], asg.at[r],
                                         asem[r])
                        pltpu.async_copy(ad_sh.at[didw.at[sp2]], adg.at[r],
                                         asem[r])
                pltpu.make_async_copy(
                    h_hbm.at[sidw.at[p]], rows.at[r], gsem[r]).wait()
                pltpu.sync_copy(wv.at[r], s_sh.at[didw.at[p]], add=True)
                if p < 3:
                    pltpu.async_copy(h_hbm.at[sidw.at[(p + 1) % 4]],
                                     rows.at[rq], gsem[rq])
                else:
                    @pl.when(g < ngrp - 1)
                    def _():
                        pltpu.async_copy(h_hbm.at[sidw.at[0]], rows.at[rq],
                                         gsem[rq])

                def srow(j, _):
                    wb = plsc.load_gather(wv.at[r], [jnp.full((16,), j, _i32)])
                    for c2 in range(D // 16):
                        sl = pl.ds(c2 * 16, 16)
                        rows[r, j, sl] = rows[r, j, sl] * wb
                    return 0

                lax.fori_loop(0, KE, srow, 0)
                pltpu.sync_copy(rows.at[r], num_sh.at[didw.at[p]], add=True)
            return 0

        lax.fori_loop(0, ngrp, step, 0)
        plsc.subcore_barrier()

        def cploop(i, _):
            rr = row0 + i * 64
            pltpu.sync_copy(num_sh.at[pl.ds(rr, 64)], rows.at[0, pl.ds(0, 64)])
            pltpu.sync_copy(rows.at[0, pl.ds(0, 64)],
                            num_out.at[cid, pl.ds(rr, 64)])
            return 0

        lax.fori_loop(0, RPT // 64, cploop, 0)
        pltpu.sync_copy(s_sh.at[pl.ds(row0, RPT)], svec)
        pltpu.sync_copy(svec, s_out.at[cid, pl.ds(row0, RPT)])

    return k(pck2d, a_s, a_d, h)


def _final_gather(table, idx3d, bpw):
    nsub = bpw // KG
    tot = NW * bpw

    @functools.partial(
        pl.kernel,
        out_type=jax.ShapeDtypeStruct((tot, D), _f32),
        mesh=_mesh(),
        scratch_types=[
            pltpu.VMEM((nsub, KG), _i32),
            pltpu.VMEM((bpw, D), _f32),
            pltpu.SemaphoreType.DMA,
        ],
    )
    def k(tab, idx, out, idx_v, rows_v, sem):
        cid = lax.axis_index("c")
        sid = lax.axis_index("s")
        wid = sid * NC + cid
        pltpu.sync_copy(idx.at[wid], idx_v)
        for j in range(nsub):
            pltpu.async_copy(tab.at[idx_v.at[j]],
                             rows_v.at[pl.ds(j * KG, KG)], sem).wait()
        pltpu.sync_copy(rows_v, out.at[pl.ds(wid * bpw, bpw)])

    return k(table, idx3d)


def _prep_edges(edge_index):
    e = edge_index.astype(_i32)
    loops = jnp.arange(N, dtype=_i32)
    src = jnp.concatenate([e[0], loops])
    dst = jnp.concatenate([e[1], loops])
    tot = src.shape[0]
    n_blk = -(-tot // (NW * KE))
    n_blk += (-n_blk) % 4
    epad = n_blk * NW * KE
    src = jnp.pad(src, (0, epad - tot))
    dst = jnp.pad(dst, (0, epad - tot), constant_values=N)
    pck = jnp.bitwise_or(src, jnp.left_shift(dst, 16))
    return pck.reshape(NW, n_blk, KE), n_blk


def kernel(embedding, edges1, edges2, idx_mapping, W, att_src, att_dst, bias):
    emb = jnp.pad(embedding, ((0, NPAD - N), (0, 0)))
    as2d = att_src.reshape(1, D)
    ad2d = att_dst.reshape(1, D)
    b2d = bias.reshape(1, D)

    p1e, nb1 = _prep_edges(edges1)
    p2e, nb2 = _prep_edges(edges2)

    h1, a_s1, a_d1 = _dense_first(emb, W, as2d, ad2d)
    num1, den1 = _edge_stage(p1e, a_s1.reshape(NPAD), a_d1.reshape(NPAD),
                             h1, nb1)
    s1r = (den1[0] + den1[1]).reshape(R, 1, BM)

    h2, a_s2, a_d2 = _dense_norm(num1, s1r, W, as2d, ad2d, b2d)
    num2, den2 = _edge_stage(p2e, a_s2.reshape(NPAD), a_d2.reshape(NPAD),
                             h2, nb2)
    s2r = (den2[0] + den2[1]).reshape(R, 1, BM)

    xf = _norm_only(num2, s2r, b2d)

    voc = idx_mapping.shape[0]
    bpw = KG * (-(-voc // (NW * KG)))
    idxp = jnp.pad(idx_mapping.astype(_i32), (0, NW * bpw - voc))
    out = _final_gather(xf, idxp.reshape(NW, bpw // KG, KG), bpw)
    return out[:voc]


# speedup vs baseline: 1.7592x; 1.7592x over previous
import functools
import jax
import jax.numpy as jnp
from jax import lax
from jax.experimental import pallas as pl
from jax.experimental.pallas import tpu as pltpu
from jax.experimental.pallas import tpu_sc as plsc

N = 10000
D = 128
NPAD = 10240
NC = 2
NS = 16
NW = NC * NS
KE = 128
KG = 128
BM = 512
R = NPAD // BM
RPT = NPAD // NS

_f32 = jnp.float32
_i32 = jnp.int32


def _mesh():
    return plsc.VectorSubcoreMesh(
        core_axis_name="c", subcore_axis_name="s", num_cores=NC, num_subcores=NS
    )


def _dense_first(x, W, att_src, att_dst):
    def body(x_ref, w_ref, asv_ref, adv_ref, h_ref, aso_ref, ado_ref):
        h = jnp.dot(x_ref[...], w_ref[...], preferred_element_type=_f32)
        h_ref[...] = h
        aso_ref[0, 0, :] = jnp.sum(h * asv_ref[0, :][None, :], axis=1)
        ado_ref[0, 0, :] = jnp.sum(h * adv_ref[0, :][None, :], axis=1)

    return pl.pallas_call(
        body,
        grid=(R,),
        in_specs=[
            pl.BlockSpec((BM, D), lambda i: (i, 0)),
            pl.BlockSpec((D, D), lambda i: (0, 0)),
            pl.BlockSpec((1, D), lambda i: (0, 0)),
            pl.BlockSpec((1, D), lambda i: (0, 0)),
        ],
        out_specs=[
            pl.BlockSpec((BM, D), lambda i: (i, 0)),
            pl.BlockSpec((1, 1, BM), lambda i: (i, 0, 0)),
            pl.BlockSpec((1, 1, BM), lambda i: (i, 0, 0)),
        ],
        out_shape=[
            jax.ShapeDtypeStruct((NPAD, D), _f32),
            jax.ShapeDtypeStruct((R, 1, BM), _f32),
            jax.ShapeDtypeStruct((R, 1, BM), _f32),
        ],
    )(x, W, att_src, att_dst)


def _dense_norm(num, s, W, att_src, att_dst, bias):
    def body(num_ref, s_ref, w_ref, asv_ref, adv_ref, b_ref,
             h_ref, aso_ref, ado_ref):
        acc = num_ref[0] + num_ref[1]
        den = s_ref[0, 0, :] + 1e-16
        x = acc / den[:, None] + b_ref[0, :][None, :]
        h = jnp.dot(x, w_ref[...], preferred_element_type=_f32)
        h_ref[...] = h
        aso_ref[0, 0, :] = jnp.sum(h * asv_ref[0, :][None, :], axis=1)
        ado_ref[0, 0, :] = jnp.sum(h * adv_ref[0, :][None, :], axis=1)

    return pl.pallas_call(
        body,
        grid=(R,),
        in_specs=[
            pl.BlockSpec((NC, BM, D), lambda i: (0, i, 0)),
            pl.BlockSpec((1, 1, BM), lambda i: (i, 0, 0)),
            pl.BlockSpec((D, D), lambda i: (0, 0)),
            pl.BlockSpec((1, D), lambda i: (0, 0)),
            pl.BlockSpec((1, D), lambda i: (0, 0)),
            pl.BlockSpec((1, D), lambda i: (0, 0)),
        ],
        out_specs=[
            pl.BlockSpec((BM, D), lambda i: (i, 0)),
            pl.BlockSpec((1, 1, BM), lambda i: (i, 0, 0)),
            pl.BlockSpec((1, 1, BM), lambda i: (i, 0, 0)),
        ],
        out_shape=[
            jax.ShapeDtypeStruct((NPAD, D), _f32),
            jax.ShapeDtypeStruct((R, 1, BM), _f32),
            jax.ShapeDtypeStruct((R, 1, BM), _f32),
        ],
    )(num, s, W, att_src, att_dst, bias)


def _norm_only(num, s, bias):
    def body(num_ref, s_ref, b_ref, x_ref):
        acc = num_ref[0] + num_ref[1]
        den = s_ref[0, 0, :] + 1e-16
        x_ref[...] = acc / den[:, None] + b_ref[0, :][None, :]

    return pl.pallas_call(
        body,
        grid=(R,),
        in_specs=[
            pl.BlockSpec((NC, BM, D), lambda i: (0, i, 0)),
            pl.BlockSpec((1, 1, BM), lambda i: (i, 0, 0)),
            pl.BlockSpec((1, D), lambda i: (0, 0)),
        ],
        out_specs=pl.BlockSpec((BM, D), lambda i: (i, 0)),
        out_shape=jax.ShapeDtypeStruct((NPAD, D), _f32),
    )(num, s, bias)


NACC = 10112
RPA = NACC // NS
NSEG = NACC // 128
SPT = -(-NSEG // NS)


def _edge_stage(pck2d, a_s, a_d, h, n_blk):
    @functools.partial(
        pl.kernel,
        out_type=(
            jax.ShapeDtypeStruct((NC, NPAD, D), _f32),
            jax.ShapeDtypeStruct((NC, NPAD), _f32),
        ),
        mesh=_mesh(),
        compiler_params=pltpu.CompilerParams(needs_layout_passes=False),
        scratch_types=[
            pltpu.VMEM((n_blk, KE), _i32),
            pltpu.VMEM((1, KE), _i32),
            pltpu.VMEM((1, KE), _i32),
            pltpu.VMEM((NPAD,), _f32),
            pltpu.VMEM((NPAD,), _f32),
            pltpu.VMEM((KE,), _f32),
            pltpu.VMEM((KE, D), _f32),
            pltpu.VMEM((128,), _f32),
            pltpu.VMEM_SHARED((NACC, D), _f32),
            pltpu.VMEM_SHARED((NACC,), _f32),
            pltpu.SemaphoreType.DMA,
        ],
    )
    def k(pck_hbm, as_hbm, ad_hbm, h_hbm, num_out, s_out,
          pck_v, sidw, didw, as_v, ad_v, w_v, rows_v, svec,
          num_sh, s_sh, sem):
        cid = lax.axis_index("c")
        sid = lax.axis_index("s")
        wid = sid * NC + cid
        row0 = sid * RPA

        pltpu.sync_copy(pck_hbm.at[wid], pck_v)
        pltpu.sync_copy(as_hbm, as_v)
        pltpu.sync_copy(ad_hbm, ad_v)

        zeros16 = jnp.zeros((16,), _f32)
        for r0 in range(64):
            for c0 in range(D // 16):
                rows_v[r0, pl.ds(c0 * 16, 16)] = zeros16
        for c0 in range(8):
            svec[pl.ds(c0 * 16, 16)] = zeros16

        zo = 0
        for ch in [64] * (RPA // 64) + ([RPA % 64] if RPA % 64 else []):
            pltpu.sync_copy(rows_v.at[pl.ds(0, ch)],
                            num_sh.at[pl.ds(row0 + zo, ch)])
            zo += ch
        for i in range(SPT):
            ss = sid * SPT + i

            @pl.when(ss < NSEG)
            def _():
                pltpu.sync_copy(svec, s_sh.at[pl.ds(ss * 128, 128)])
        plsc.subcore_barrier()

        def blk(b, _):
            for c2 in range(KE // 16):
                sl = pl.ds(c2 * 16, 16)
                wrd = pck_v[b, sl]
                si = jnp.bitwise_and(wrd, 0xFFFF)
                di = lax.shift_right_logical(wrd, 16)
                sidw[0, sl] = si
                didw[0, sl] = di
                e = plsc.load_gather(as_v, [si]) + plsc.load_gather(ad_v, [di])
                e = jnp.where(e >= 0.0, e, 0.2 * e)
                w_v[sl] = jnp.exp(e)
            pltpu.sync_copy(w_v, s_sh.at[didw.at[0]], add=True)
            pltpu.async_copy(h_hbm.at[sidw.at[0]], rows_v, sem).wait()

            def srow(j4, _):
                j = j4 * 4
                for dj in range(4):
                    wb = plsc.load_gather(
                        w_v, [jnp.full((16,), j + dj, _i32)])
                    for c2 in range(D // 16):
                        sl = pl.ds(c2 * 16, 16)
                        rows_v[j + dj, sl] = rows_v[j + dj, sl] * wb
                return 0

            lax.fori_loop(0, KE // 4, srow, 0)
            pltpu.sync_copy(rows_v, num_sh.at[didw.at[0]], add=True)
            return 0

        lax.fori_loop(0, n_blk, blk, 0)
        plsc.subcore_barrier()

        co = 0
        for ch in [64] * (RPA // 64) + ([RPA % 64] if RPA % 64 else []):
            r = row0 + co
            pltpu.sync_copy(num_sh.at[pl.ds(r, ch)], rows_v.at[pl.ds(0, ch)])
            pltpu.sync_copy(rows_v.at[pl.ds(0, ch)],
                            num_out.at[cid, pl.ds(r, ch)])
            co += ch
        for i in range(SPT):
            ss = sid * SPT + i

            @pl.when(ss < NSEG)
            def _():
                pltpu.sync_copy(s_sh.at[pl.ds(ss * 128, 128)], svec)
                pltpu.sync_copy(svec, s_out.at[cid, pl.ds(ss * 128, 128)])

    return k(pck2d, a_s, a_d, h)


def _final_gather(table, idx3d, bpw):
    nsub = bpw // KG
    tot = NW * bpw

    @functools.partial(
        pl.kernel,
        out_type=jax.ShapeDtypeStruct((tot, D), _f32),
        mesh=_mesh(),
        scratch_types=[
            pltpu.VMEM((nsub, KG), _i32),
            pltpu.VMEM((bpw, D), _f32),
            pltpu.SemaphoreType.DMA,
        ],
    )
    def k(tab, idx, out, idx_v, rows_v, sem):
        cid = lax.axis_index("c")
        sid = lax.axis_index("s")
        wid = sid * NC + cid
        pltpu.sync_copy(idx.at[wid], idx_v)
        for j in range(nsub):
            pltpu.async_copy(tab.at[idx_v.at[j]],
                             rows_v.at[pl.ds(j * KG, KG)], sem).wait()
        pltpu.sync_copy(rows_v, out.at[pl.ds(wid * bpw, bpw)])

    return k(table, idx3d)


def _prep_edges(edge_index):
    e = edge_index.astype(_i32)
    loops = jnp.arange(N, dtype=_i32)
    src = jnp.concatenate([e[0], loops])
    dst = jnp.concatenate([e[1], loops])
    tot = src.shape[0]
    n_blk = -(-tot // (NW * KE))
    epad = n_blk * NW * KE
    src = jnp.pad(src, (0, epad - tot))
    dst = jnp.pad(dst, (0, epad - tot), constant_values=N)
    pck = jnp.bitwise_or(src, jnp.left_shift(dst, 16))
    return pck.reshape(NW, n_blk, KE), n_blk


def kernel(embedding, edges1, edges2, idx_mapping, W, att_src, att_dst, bias):
    emb = jnp.pad(embedding, ((0, NPAD - N), (0, 0)))
    as2d = att_src.reshape(1, D)
    ad2d = att_dst.reshape(1, D)
    b2d = bias.reshape(1, D)

    p1e, nb1 = _prep_edges(edges1)
    p2e, nb2 = _prep_edges(edges2)

    h1, a_s1, a_d1 = _dense_first(emb, W, as2d, ad2d)
    num1, den1 = _edge_stage(p1e, a_s1.reshape(NPAD), a_d1.reshape(NPAD),
                             h1, nb1)
    s1r = (den1[0] + den1[1]).reshape(R, 1, BM)

    h2, a_s2, a_d2 = _dense_norm(num1, s1r, W, as2d, ad2d, b2d)
    num2, den2 = _edge_stage(p2e, a_s2.reshape(NPAD), a_d2.reshape(NPAD),
                             h2, nb2)
    s2r = (den2[0] + den2[1]).reshape(R, 1, BM)

    xf = _norm_only(num2, s2r, b2d)

    voc = idx_mapping.shape[0]
    bpw = KG * (-(-voc // (NW * KG)))
    idxp = jnp.pad(idx_mapping.astype(_i32), (0, NW * bpw - voc))
    out = _final_gather(xf, idxp.reshape(NW, bpw // KG, KG), bpw)
    return out[:voc]
